# baseline (speedup 1.0000x reference)
import functools

import jax
import jax.numpy as jnp
from jax import lax
from jax.experimental import pallas as pl
from jax.experimental.pallas import tpu as pltpu
from jax.experimental.pallas import tpu_sc as plsc

_NC, _NS = 2, 16
_NW = _NC * _NS
_B, _NEG, _D = 1024, 200, 128
_TAIL = _B * (_NEG + 1)
_RPW = _TAIL // _NW
_CH = 96
_NCHUNK = _RPW // _CH
_NB = 4
_NCHUNK_PAD = 68
_HPW = _B // _NW


def _sc_gather(entity, relation, head_idx, rel_idx, tail_idx):
    mesh = plsc.VectorSubcoreMesh(core_axis_name="c", subcore_axis_name="s")

    @functools.partial(
        pl.kernel,
        mesh=mesh,
        out_type=[
            jax.ShapeDtypeStruct((_B, _D), jnp.float32),
            jax.ShapeDtypeStruct((_B, _D), jnp.float32),
            jax.ShapeDtypeStruct((_TAIL, _D), jnp.float32),
        ],
        scratch_types=[
            pltpu.VMEM((_HPW,), jnp.int32),
            pltpu.VMEM((_HPW, _D), jnp.float32),
            pltpu.VMEM((_NCHUNK_PAD, _CH), jnp.int32),
        ] + [pltpu.VMEM((_CH, _D), jnp.float32) for _ in range(_NB)]
          + [pltpu.SemaphoreType.DMA for _ in range(2 * _NB)],
    )
    def k(ent_hbm, rel_hbm, hidx_hbm, ridx_hbm, tidx_hbm,
          head_out, rel_out, tail_out,
          sidx_v, srow_v, tidx_v, *bufs_and_sems):
        rows = bufs_and_sems[:_NB]
        gsem = bufs_and_sems[_NB:2 * _NB]
        ssem = bufs_and_sems[2 * _NB:]
        wid = lax.axis_index("s") * _NC + lax.axis_index("c")

        hbase = wid * _HPW
        pltpu.sync_copy(hidx_hbm.at[wid], sidx_v)
        pltpu.async_copy(ent_hbm.at[sidx_v], srow_v, gsem[0]).wait()
        pltpu.sync_copy(srow_v, head_out.at[pl.ds(hbase, _HPW)])

        pltpu.sync_copy(ridx_hbm.at[wid], sidx_v)
        pltpu.async_copy(rel_hbm.at[sidx_v], srow_v, gsem[0]).wait()
        pltpu.sync_copy(srow_v, rel_out.at[pl.ds(hbase, _HPW)])

        tbase = wid * _RPW
        pltpu.sync_copy(tidx_hbm.at[wid], tidx_v)

        def gather_start(c, b):
            pltpu.async_copy(ent_hbm.at[tidx_v.at[c]], rows[b], gsem[b])

        def gather_wait(b):
            pltpu.make_async_copy(
                ent_hbm.at[tidx_v.at[0]], rows[b], gsem[b]).wait()

        def store_start(c, b):
            pltpu.async_copy(
                rows[b], tail_out.at[pl.ds(tbase + c * _CH, _CH)], ssem[b])

        def store_wait(b):
            pltpu.make_async_copy(
                rows[b], tail_out.at[pl.ds(tbase, _CH)], ssem[b]).wait()

        for b in range(_NB):
            gather_start(b, b)

        def body(g, carry):
            for b in range(_NB):
                gather_wait(b)
                store_start(g * _NB + b, b)
            for b in range(_NB):
                store_wait(b)
                gather_start((g + 1) * _NB + b, b)
            return carry

        lax.fori_loop(0, _NCHUNK_PAD // _NB - 1, body, 0)

        last = _NCHUNK_PAD - _NB
        for b in range(_NB):
            gather_wait(b)
            if last + b < _NCHUNK:
                pltpu.sync_copy(
                    rows[b], tail_out.at[pl.ds(tbase + (last + b) * _CH, _CH)])

    return k(entity, relation, head_idx, rel_idx, tail_idx)


def kernel(positive, negative, entity_embedding, relation_embedding):
    positive = positive.astype(jnp.int32)
    negative = negative.astype(jnp.int32)
    head_idx = positive[:, 0].reshape(_NW, _HPW)
    rel_idx = positive[:, 1].reshape(_NW, _HPW)
    tail_idx = jnp.concatenate(
        [positive[:, 2:3], negative], axis=1).reshape(_NW, _RPW)
    pad = _NCHUNK_PAD * _CH - _RPW
    tail_idx = jnp.pad(tail_idx, ((0, 0), (0, pad))).reshape(
        _NW, _NCHUNK_PAD, _CH)
    head, rel, tail = _sc_gather(
        entity_embedding, relation_embedding, head_idx, rel_idx, tail_idx)
    return (head[:, None, :], rel[:, None, :], tail.reshape(_B, _NEG + 1, _D))
